# Initial kernel scaffold; baseline (speedup 1.0000x reference)
import jax
import jax.numpy as jnp
from jax.experimental import pallas as pl


def kernel(x, edge_index, W, a):
    raise NotImplementedError("write your pallas kernel here")


# speedup vs baseline: 7.9823x; 7.9823x over previous
import functools

import jax
import jax.numpy as jnp
from jax import lax
from jax.experimental import pallas as pl
from jax.experimental.pallas import tpu as pltpu
from jax.experimental.pallas import tpu_sc as plsc

_N = 10000
_F = 128
_E = 330000
_G = _E // 16
_NW = 32
_GPW = 648
_CMAX = _GPW
_GPAD = _GPW * _NW
_NROW = 10240
_RPT = _NROW // 16
_NPAD = 10112


def _prep_body(x_ref, w_ref, a_ref, act_ref, ss_ref, sd_ref):
    act = lax.dot_general(x_ref[...], w_ref[...], (((1,), (1,)), ((), ())),
                          preferred_element_type=jnp.float32)
    act_ref[...] = act
    ss_ref[...] = jnp.sum(act * a_ref[0][None, :], axis=1, keepdims=True)
    sd_ref[...] = jnp.sum(act * a_ref[1][None, :], axis=1, keepdims=True)


def _prep(x, W, a2):
    return pl.pallas_call(
        _prep_body,
        grid=(10,),
        in_specs=[
            pl.BlockSpec((1000, _F), lambda i: (i, 0)),
            pl.BlockSpec((_F, _F), lambda i: (0, 0)),
            pl.BlockSpec((2, _F), lambda i: (0, 0)),
        ],
        out_specs=[
            pl.BlockSpec((1000, _F), lambda i: (i, 0)),
            pl.BlockSpec((1000, 1), lambda i: (i, 0)),
            pl.BlockSpec((1000, 1), lambda i: (i, 0)),
        ],
        out_shape=[
            jax.ShapeDtypeStruct((_N, _F), jnp.float32),
            jax.ShapeDtypeStruct((_N, 1), jnp.float32),
            jax.ShapeDtypeStruct((_N, 1), jnp.float32),
        ],
    )(x, W, a2)


_mesh = plsc.VectorSubcoreMesh(
    core_axis_name="c", subcore_axis_name="s", num_cores=2, num_subcores=16
)


@functools.partial(
    pl.kernel,
    out_type=[
        jax.ShapeDtypeStruct((_E, _F), jnp.float32),
        jax.ShapeDtypeStruct((_GPAD * 16,), jnp.float32),
        jax.ShapeDtypeStruct((2, _NROW, _F), jnp.float32),
        jax.ShapeDtypeStruct((2, _NROW), jnp.float32),
    ],
    mesh=_mesh,
    scratch_types=[
        pltpu.VMEM((_CMAX, 16), jnp.int32),
        pltpu.VMEM((_CMAX, 16), jnp.int32),
        pltpu.VMEM((_CMAX * 16,), jnp.float32),
        pltpu.VMEM((16,), jnp.float32),
        pltpu.VMEM((16,), jnp.float32),
        pltpu.VMEM((16,), jnp.float32),
        pltpu.VMEM((16,), jnp.float32),
        pltpu.VMEM((16, _F), jnp.float32),
        pltpu.VMEM((16, _F), jnp.float32),
        pltpu.VMEM((16, _F), jnp.float32),
        pltpu.VMEM((16, _F), jnp.float32),
        pltpu.VMEM((_RPT,), jnp.float32),
        pltpu.VMEM_SHARED((_NROW, _F), jnp.float32),
        pltpu.VMEM_SHARED((_NROW,), jnp.float32),
        pltpu.SemaphoreType.DMA,
        pltpu.SemaphoreType.DMA,
        pltpu.SemaphoreType.DMA,
        pltpu.SemaphoreType.DMA,
        pltpu.SemaphoreType.DMA,
        pltpu.SemaphoreType.DMA,
        pltpu.SemaphoreType.DMA,
        pltpu.SemaphoreType.DMA,
    ],
    compiler_params=pltpu.CompilerParams(
        needs_layout_passes=False, use_tc_tiling_on_sc=False
    ),
)
def _sc_main(act, src2, dst2, ssrc, sdst,
             msg_out, w_out, agg_out, den_out,
             sidx, didx, wout, ssb0, ssb1, sdb0, sdb1,
             mb0, mb1, wm0, wm1, zb,
             agg_sh, den_sh, sg0, sg1, sw0, sw1, sa0, sa1, sd0, sd1):
    cid = lax.axis_index("c")
    sid = lax.axis_index("s")
    wid = cid * 16 + sid
    g_start = wid * _GPW
    g_cnt = jnp.minimum(_GPW, _G - g_start)

    pltpu.sync_copy(src2.at[pl.ds(g_start, _CMAX)], sidx)
    pltpu.sync_copy(dst2.at[pl.ds(g_start, _CMAX)], didx)

    zero16 = jnp.zeros((16,), jnp.float32)
    for r in range(16):
        for c in range(8):
            wm0[r, pl.ds(c * 16, 16)] = zero16
    for k in range(_RPT // 16):
        zb[pl.ds(k * 16, 16)] = zero16
    base = sid * _RPT
    for k in range(_RPT // 16):
        pltpu.sync_copy(wm0, agg_sh.at[pl.ds(base + k * 16, 16)])
    pltpu.sync_copy(zb, den_sh.at[pl.ds(base, _RPT)])
    plsc.subcore_barrier()

    pltpu.async_copy(act.at[sidx.at[0]], mb0, sg0)
    pltpu.async_copy(ssrc.at[sidx.at[0]], ssb0, sg0)
    pltpu.async_copy(sdst.at[didx.at[0]], sdb0, sg0)
    pltpu.async_copy(act.at[sidx.at[1]], mb1, sg1)
    pltpu.async_copy(ssrc.at[sidx.at[1]], ssb1, sg1)
    pltpu.async_copy(sdst.at[didx.at[1]], sdb1, sg1)

    def chunk_body(j, mb, wm, ssb, sdb, sg, sw, sa, sdn):
        e0 = (g_start + j) * 16
        pltpu.make_async_copy(act.at[sidx.at[j]], mb, sg).wait()
        pltpu.make_async_copy(ssrc.at[sidx.at[j]], ssb, sg).wait()
        pltpu.make_async_copy(sdst.at[didx.at[j]], sdb, sg).wait()
        logit = ssb[...] + sdb[...]
        w16 = jnp.exp(jnp.maximum(logit, logit * jnp.float32(0.01)))
        wout[pl.ds(j * 16, 16)] = w16
        mw = pltpu.async_copy(mb, msg_out.at[pl.ds(e0, 16)], sw)

        @pl.when(j >= 2)
        def _():
            pltpu.make_async_copy(wm, agg_sh.at[didx.at[j]], sa).wait()
            pltpu.make_async_copy(wout.at[pl.ds(0, 16)],
                                  den_sh.at[didx.at[j]], sdn).wait()

        for r in range(16):
            wsp = plsc.load_gather(wout, [jnp.full((16,), j * 16 + r, jnp.int32)])
            for c in range(8):
                wm[r, pl.ds(c * 16, 16)] = mb[r, pl.ds(c * 16, 16)] * wsp
        pltpu.async_copy(wm, agg_sh.at[didx.at[j]], sa, add=True)
        pltpu.async_copy(wout.at[pl.ds(j * 16, 16)], den_sh.at[didx.at[j]],
                         sdn, add=True)
        mw.wait()

        @pl.when(j + 2 < g_cnt)
        def _():
            pltpu.async_copy(act.at[sidx.at[j + 2]], mb, sg)
            pltpu.async_copy(ssrc.at[sidx.at[j + 2]], ssb, sg)
            pltpu.async_copy(sdst.at[didx.at[j + 2]], sdb, sg)

    def outer(j2, carry):
        j0 = j2 * 2
        chunk_body(j0, mb0, wm0, ssb0, sdb0, sg0, sw0, sa0, sd0)

        @pl.when(j0 + 1 < g_cnt)
        def _():
            chunk_body(j0 + 1, mb1, wm1, ssb1, sdb1, sg1, sw1, sa1, sd1)

        return carry

    lax.fori_loop(0, (g_cnt + 1) // 2, outer, 0)

    pltpu.make_async_copy(wm0, agg_sh.at[didx.at[0]], sa0).wait()
    pltpu.make_async_copy(wout.at[pl.ds(0, 16)], den_sh.at[didx.at[0]], sd0).wait()
    pltpu.make_async_copy(wm1, agg_sh.at[didx.at[0]], sa1).wait()
    pltpu.make_async_copy(wout.at[pl.ds(0, 16)], den_sh.at[didx.at[0]], sd1).wait()

    pltpu.sync_copy(wout, w_out.at[pl.ds(g_start * 16, _GPW * 16)])

    plsc.subcore_barrier()
    pltpu.sync_copy(agg_sh.at[pl.ds(base, _RPT)],
                    agg_out.at[cid, pl.ds(base, _RPT)])
    pltpu.sync_copy(den_sh.at[pl.ds(base, _RPT)],
                    den_out.at[cid, pl.ds(base, _RPT)])


def _fin_body(agg_ref, den_ref, out_ref, dout_ref):
    d = den_ref[0] + den_ref[1]
    out_ref[...] = (agg_ref[0] + agg_ref[1]) / d[:, None]
    dout_ref[...] = d[:, None]


def _fin(agg_p, den_p):
    return pl.pallas_call(
        _fin_body,
        grid=(8,),
        in_specs=[
            pl.BlockSpec((2, 1280, _F), lambda i: (0, i, 0)),
            pl.BlockSpec((2, 1280), lambda i: (0, i)),
        ],
        out_specs=[
            pl.BlockSpec((1280, _F), lambda i: (i, 0)),
            pl.BlockSpec((1280, 1), lambda i: (i, 0)),
        ],
        out_shape=[
            jax.ShapeDtypeStruct((_N, _F), jnp.float32),
            jax.ShapeDtypeStruct((_N, 1), jnp.float32),
        ],
    )(agg_p, den_p)


def kernel(x, edge_index, W, a):
    ei = edge_index.astype(jnp.int32)
    loops = jnp.arange(_N, dtype=jnp.int32)
    src = jnp.concatenate([ei[0], loops])
    dst = jnp.concatenate([ei[1], loops])
    src2 = jnp.pad(src.reshape(_G, 16), ((0, _GPAD - _G), (0, 0)))
    dst2 = jnp.pad(dst.reshape(_G, 16), ((0, _GPAD - _G), (0, 0)))
    act, ss, sd = _prep(x, W, a.reshape(2, _F))
    ssp = jnp.pad(ss[:, 0], (0, _NPAD - _N))
    sdp = jnp.pad(sd[:, 0], (0, _NPAD - _N))
    msg, w_pad, agg_p, den_p = _sc_main(act, src2, dst2, ssp, sdp)
    aggregated, den = _fin(agg_p, den_p)
    return (aggregated, w_pad[:_E], den[:, 0], msg)
